# baseline (speedup 1.0000x reference)
import jax
import jax.numpy as jnp
from jax.experimental import pallas as pl

B, M, D, HOPS = 32, 4096, 128, 3


def _body(q_ref, g_ref, m0_ref, m1_ref, m2_ref, soft_ref, logits_ref):
    u = q_ref[0, 0, :]
    g = g_ref[0, 0, :]
    a0 = m0_ref[0]
    a1 = m1_ref[0]
    a2 = m2_ref[0]

    def hop(a_logits, a_next, u):
        l = jnp.sum(a_logits * u[None, :], axis=1) * g
        p = jax.nn.softmax(l)
        w = p * g
        o = jnp.sum(a_next * w[:, None], axis=0)
        return l, p, u + o

    _, _, u = hop(a0, a1, u)
    _, _, u = hop(a1, a2, u)
    l = jnp.sum(a2 * u[None, :], axis=1) * g
    p = jax.nn.softmax(l)
    soft_ref[0, 0, :] = p
    logits_ref[0, 0, :] = l


@jax.jit
def kernel(query_vector, global_pointer, m0, m1, m2, m3):
    del m3
    out = pl.pallas_call(
        _body,
        grid=(B,),
        in_specs=[
            pl.BlockSpec((1, 1, D), lambda i: (i, 0, 0)),
            pl.BlockSpec((1, 1, M), lambda i: (i, 0, 0)),
            pl.BlockSpec((1, M, D), lambda i: (i, 0, 0)),
            pl.BlockSpec((1, M, D), lambda i: (i, 0, 0)),
            pl.BlockSpec((1, M, D), lambda i: (i, 0, 0)),
        ],
        out_specs=[
            pl.BlockSpec((1, 1, M), lambda i: (i, 0, 0)),
            pl.BlockSpec((1, 1, M), lambda i: (i, 0, 0)),
        ],
        out_shape=[
            jax.ShapeDtypeStruct((B, 1, M), jnp.float32),
            jax.ShapeDtypeStruct((B, 1, M), jnp.float32),
        ],
    )(query_vector[:, None, :], global_pointer[:, None, :], m0, m1, m2)
    return (out[0][:, 0, :], out[1][:, 0, :])


# speedup vs baseline: 3.9391x; 3.9391x over previous
import jax
import jax.numpy as jnp
from jax.experimental import pallas as pl

B, M, D, HOPS = 32, 4096, 128, 3


def _logits(a, u, g):
    t = jax.lax.dot_general(u, a, (((1,), (1,)), ((), ())),
                            preferred_element_type=jnp.float32)
    return t * g


def _softmax_row(l):
    m = jnp.max(l, axis=1, keepdims=True)
    e = jnp.exp(l - m)
    return e / jnp.sum(e, axis=1, keepdims=True)


def _body(q_ref, g_ref, m0_ref, m1_ref, m2_ref, soft_ref, logits_ref):
    u = q_ref[0]
    g = g_ref[0]
    a0 = m0_ref[0]
    a1 = m1_ref[0]
    a2 = m2_ref[0]

    def hop(a_logits, a_next, u):
        l = _logits(a_logits, u, g)
        p = _softmax_row(l)
        w = p * g
        o = jax.lax.dot_general(w, a_next, (((1,), (0,)), ((), ())),
                                preferred_element_type=jnp.float32)
        return l, p, u + o

    _, _, u = hop(a0, a1, u)
    _, _, u = hop(a1, a2, u)
    l = _logits(a2, u, g)
    p = _softmax_row(l)
    soft_ref[0] = p
    logits_ref[0] = l


@jax.jit
def kernel(query_vector, global_pointer, m0, m1, m2, m3):
    del m3
    out = pl.pallas_call(
        _body,
        grid=(B,),
        in_specs=[
            pl.BlockSpec((1, 1, D), lambda i: (i, 0, 0)),
            pl.BlockSpec((1, 1, M), lambda i: (i, 0, 0)),
            pl.BlockSpec((1, M, D), lambda i: (i, 0, 0)),
            pl.BlockSpec((1, M, D), lambda i: (i, 0, 0)),
            pl.BlockSpec((1, M, D), lambda i: (i, 0, 0)),
        ],
        out_specs=[
            pl.BlockSpec((1, 1, M), lambda i: (i, 0, 0)),
            pl.BlockSpec((1, 1, M), lambda i: (i, 0, 0)),
        ],
        out_shape=[
            jax.ShapeDtypeStruct((B, 1, M), jnp.float32),
            jax.ShapeDtypeStruct((B, 1, M), jnp.float32),
        ],
    )(query_vector[:, None, :], global_pointer[:, None, :], m0, m1, m2)
    return (out[0][:, 0, :], out[1][:, 0, :])


# speedup vs baseline: 4.4894x; 1.1397x over previous
import jax
import jax.numpy as jnp
from jax.experimental import pallas as pl

B, M, D, HOPS = 32, 4096, 128, 3
BSZ = 2


def _logits(a, u, g):
    t = jax.lax.dot_general(u, a, (((1,), (1,)), ((), ())),
                            preferred_element_type=jnp.float32)
    return t * g


def _body(q_ref, g_ref, m0_ref, m1_ref, m2_ref, soft_ref, logits_ref):
    for b in range(BSZ):
        u = q_ref[0, b][None, :]
        g = g_ref[0, b][None, :]
        a0 = m0_ref[b]
        a1 = m1_ref[b]
        a2 = m2_ref[b]

        def hop(a_logits, a_next, u, g):
            l = _logits(a_logits, u, g)
            e = jnp.exp(l - jnp.max(l, axis=1, keepdims=True))
            eg = e * g
            o = jax.lax.dot_general(eg, a_next, (((1,), (0,)), ((), ())),
                                    preferred_element_type=jnp.float32)
            return u + o / jnp.sum(e, axis=1, keepdims=True)

        u = hop(a0, a1, u, g)
        u = hop(a1, a2, u, g)
        l = _logits(a2, u, g)
        e = jnp.exp(l - jnp.max(l, axis=1, keepdims=True))
        p = e / jnp.sum(e, axis=1, keepdims=True)
        soft_ref[0, b] = p[0]
        logits_ref[0, b] = l[0]


@jax.jit
def kernel(query_vector, global_pointer, m0, m1, m2, m3):
    del m3
    out = pl.pallas_call(
        _body,
        grid=(B // BSZ,),
        in_specs=[
            pl.BlockSpec((1, BSZ, D), lambda i: (i, 0, 0)),
            pl.BlockSpec((1, BSZ, M), lambda i: (i, 0, 0)),
            pl.BlockSpec((BSZ, M, D), lambda i: (i, 0, 0)),
            pl.BlockSpec((BSZ, M, D), lambda i: (i, 0, 0)),
            pl.BlockSpec((BSZ, M, D), lambda i: (i, 0, 0)),
        ],
        out_specs=[
            pl.BlockSpec((1, BSZ, M), lambda i: (i, 0, 0)),
            pl.BlockSpec((1, BSZ, M), lambda i: (i, 0, 0)),
        ],
        out_shape=[
            jax.ShapeDtypeStruct((B // BSZ, BSZ, M), jnp.float32),
            jax.ShapeDtypeStruct((B // BSZ, BSZ, M), jnp.float32),
        ],
    )(query_vector.reshape(B // BSZ, BSZ, D),
      global_pointer.reshape(B // BSZ, BSZ, M), m0, m1, m2)
    return (out[0].reshape(B, M), out[1].reshape(B, M))


# speedup vs baseline: 4.6374x; 1.0330x over previous
import jax
import jax.numpy as jnp
from jax.experimental import pallas as pl

B, M, D, HOPS = 32, 4096, 128, 3
BSZ = 4


def _logits(a, u, g):
    t = jax.lax.dot_general(u, a, (((1,), (1,)), ((), ())),
                            preferred_element_type=jnp.float32)
    return t * g


def _body(q_ref, g_ref, m0_ref, m1_ref, m2_ref, soft_ref, logits_ref):
    for b in range(BSZ):
        u = q_ref[0, b][None, :]
        g = g_ref[0, b][None, :]
        a0 = m0_ref[b]
        a1 = m1_ref[b]
        a2 = m2_ref[b]

        def hop(a_logits, a_next, u, g):
            l = _logits(a_logits, u, g)
            e = jnp.exp(l - jnp.max(l, axis=1, keepdims=True))
            eg = e * g
            o = jax.lax.dot_general(eg, a_next, (((1,), (0,)), ((), ())),
                                    preferred_element_type=jnp.float32)
            return u + o / jnp.sum(e, axis=1, keepdims=True)

        u = hop(a0, a1, u, g)
        u = hop(a1, a2, u, g)
        l = _logits(a2, u, g)
        e = jnp.exp(l - jnp.max(l, axis=1, keepdims=True))
        p = e / jnp.sum(e, axis=1, keepdims=True)
        soft_ref[0, b] = p[0]
        logits_ref[0, b] = l[0]


@jax.jit
def kernel(query_vector, global_pointer, m0, m1, m2, m3):
    del m3
    out = pl.pallas_call(
        _body,
        grid=(B // BSZ,),
        in_specs=[
            pl.BlockSpec((1, BSZ, D), lambda i: (i, 0, 0)),
            pl.BlockSpec((1, BSZ, M), lambda i: (i, 0, 0)),
            pl.BlockSpec((BSZ, M, D), lambda i: (i, 0, 0)),
            pl.BlockSpec((BSZ, M, D), lambda i: (i, 0, 0)),
            pl.BlockSpec((BSZ, M, D), lambda i: (i, 0, 0)),
        ],
        out_specs=[
            pl.BlockSpec((1, BSZ, M), lambda i: (i, 0, 0)),
            pl.BlockSpec((1, BSZ, M), lambda i: (i, 0, 0)),
        ],
        out_shape=[
            jax.ShapeDtypeStruct((B // BSZ, BSZ, M), jnp.float32),
            jax.ShapeDtypeStruct((B // BSZ, BSZ, M), jnp.float32),
        ],
    )(query_vector.reshape(B // BSZ, BSZ, D),
      global_pointer.reshape(B // BSZ, BSZ, M), m0, m1, m2)
    return (out[0].reshape(B, M), out[1].reshape(B, M))
